# baseline (speedup 1.0000x reference)
import functools

import jax
import jax.numpy as jnp
from jax import lax
from jax.experimental import pallas as pl
from jax.experimental.pallas import tpu as pltpu
from jax.experimental.pallas import tpu_sc as plsc

_LANES = 16
_SUBCORES = 16


def _build_sc_kernel(B, S, D):
    mesh = plsc.VectorSubcoreMesh(
        core_axis_name="c", subcore_axis_name="s", num_cores=1
    )
    group = _SUBCORES // B
    seg = S // group

    @functools.partial(
        pl.kernel,
        mesh=mesh,
        out_type=jax.ShapeDtypeStruct((B, D), jnp.float32),
        scratch_types=[
            pltpu.VMEM((seg,), jnp.float32),
            pltpu.SMEM((1,), jnp.int32),
        ],
    )
    def k(do_hbm, mask_hbm, out_hbm, mask_v, total_sm):
        s = lax.axis_index("s")
        b = s // group
        q = s % group
        agg = b * group

        pltpu.sync_copy(mask_hbm.at[b, pl.ds(q * seg, seg)], mask_v)

        unroll = 16
        span = unroll * _LANES

        def body(i, acc):
            base = i * span
            for j in range(unroll):
                acc = acc + mask_v[pl.ds(base + j * _LANES, _LANES)]
            return acc

        acc = lax.fori_loop(
            0, seg // span, body, jnp.zeros((_LANES,), jnp.float32)
        )
        acc_i = acc.astype(jnp.int32)
        partial = acc_i[0]
        for j in range(1, _LANES):
            partial = partial + acc_i[j]

        @pl.when(q == 0)
        def _():
            total_sm[0] = partial

        plsc.subcore_barrier()

        @pl.when(q != 0)
        def _():
            plsc.fetch_and_add(total_sm.at[0], partial, subcore_id=agg)

        plsc.subcore_barrier()

        @pl.when(q == 0)
        def _():
            idx = jnp.maximum(total_sm[0] - 1, 0)
            row = b * S + idx
            pltpu.sync_copy(do_hbm.at[pl.ds(row, 1)], out_hbm.at[pl.ds(b, 1)])

    return k


@jax.jit
def kernel(decoder_outputs, attention_mask):
    B, S, D = decoder_outputs.shape
    do2d = decoder_outputs.reshape(B * S, D)
    k = _build_sc_kernel(B, S, D)
    return k(do2d, attention_mask.astype(jnp.float32))
